# baseline (speedup 1.0000x reference)
import jax
import jax.numpy as jnp
from jax.experimental import pallas as pl

N = 4096
C = 16
RA = 1024
RB = 1024


def _cast_body(l_ref, x_ref, y_ref, lb_ref):
    l = l_ref[...]
    y_ref[...] = jnp.dot(l, x_ref[...], preferred_element_type=jnp.float32)
    lb_ref[...] = l.astype(jnp.bfloat16)


def _cast_stage(L, x):
    return pl.pallas_call(
        _cast_body,
        grid=(N // RA,),
        in_specs=[
            pl.BlockSpec((RA, N), lambda i: (i, 0)),
            pl.BlockSpec((N, C), lambda i: (0, 0)),
        ],
        out_specs=[
            pl.BlockSpec((RA, C), lambda i: (i, 0)),
            pl.BlockSpec((RA, N), lambda i: (i, 0)),
        ],
        out_shape=[
            jax.ShapeDtypeStruct((N, C), jnp.float32),
            jax.ShapeDtypeStruct((N, N), jnp.bfloat16),
        ],
    )(L, x)


def _pair_body(ld_ref, lu_ref, x_ref, yd_ref, yu_ref):
    x = x_ref[...].astype(jnp.bfloat16)
    yd_ref[...] = jnp.dot(ld_ref[...], x, preferred_element_type=jnp.float32)
    yu_ref[...] = jnp.dot(lu_ref[...], x, preferred_element_type=jnp.float32)


def _pair_stage(Ldb, Lub, h):
    return pl.pallas_call(
        _pair_body,
        grid=(N // RB,),
        in_specs=[
            pl.BlockSpec((RB, N), lambda i: (i, 0)),
            pl.BlockSpec((RB, N), lambda i: (i, 0)),
            pl.BlockSpec((N, C), lambda i: (0, 0)),
        ],
        out_specs=[
            pl.BlockSpec((RB, C), lambda i: (i, 0)),
            pl.BlockSpec((RB, C), lambda i: (i, 0)),
        ],
        out_shape=[
            jax.ShapeDtypeStruct((N, C), jnp.float32),
            jax.ShapeDtypeStruct((N, C), jnp.float32),
        ],
    )(Ldb, Lub, h)


def _combine_body(ld_ref, lu_ref, xd_ref, xu_ref, x0_ref, g_ref, b_ref,
                  out_ref):
    i = pl.program_id(0)
    rows = pl.ds(i * RB, RB)
    xd = xd_ref[...].astype(jnp.bfloat16)
    xu = xu_ref[...].astype(jnp.bfloat16)
    xd2 = jnp.dot(ld_ref[...], xd, preferred_element_type=jnp.float32)
    xu2 = jnp.dot(lu_ref[...], xu, preferred_element_type=jnp.float32)
    acc = jnp.dot(x0_ref[rows, :], g_ref[0], preferred_element_type=jnp.float32)
    acc += jnp.dot(xd_ref[rows, :], g_ref[1], preferred_element_type=jnp.float32)
    acc += jnp.dot(xd2, g_ref[2], preferred_element_type=jnp.float32)
    acc += jnp.dot(xu_ref[rows, :], g_ref[3], preferred_element_type=jnp.float32)
    acc += jnp.dot(xu2, g_ref[4], preferred_element_type=jnp.float32)
    out_ref[...] = acc + b_ref[...]


def _combine_stage(Ldb, Lub, xd1, xu1, x0, G, b):
    return pl.pallas_call(
        _combine_body,
        grid=(N // RB,),
        in_specs=[
            pl.BlockSpec((RB, N), lambda i: (i, 0)),
            pl.BlockSpec((RB, N), lambda i: (i, 0)),
            pl.BlockSpec((N, C), lambda i: (0, 0)),
            pl.BlockSpec((N, C), lambda i: (0, 0)),
            pl.BlockSpec((N, C), lambda i: (0, 0)),
            pl.BlockSpec((5, C, C), lambda i: (0, 0, 0)),
            pl.BlockSpec((1, C), lambda i: (0, 0)),
        ],
        out_specs=pl.BlockSpec((RB, C), lambda i: (i, 0)),
        out_shape=jax.ShapeDtypeStruct((N, C), jnp.float32),
    )(Ldb, Lub, xd1, xu1, x0, G, b)


def kernel(x, laplacian_down, laplacian_up, W1, W2, W_lin, b_lin):
    G1 = jnp.transpose(W1, (2, 0, 1))
    V2 = jnp.einsum("iok,oj->kij", W2, W_lin)
    zero_b = jnp.zeros((1, C), jnp.float32)
    b2 = b_lin.reshape(1, C).astype(jnp.float32)

    xd1, Ldb = _cast_stage(laplacian_down, x)
    xu1, Lub = _cast_stage(laplacian_up, x)
    h = _combine_stage(Ldb, Lub, xd1, xu1, x, G1, zero_b)
    hd1, hu1 = _pair_stage(Ldb, Lub, h)
    out = _combine_stage(Ldb, Lub, hd1, hu1, h, V2, b2)
    return out


# speedup vs baseline: 1.0068x; 1.0068x over previous
import jax
import jax.numpy as jnp
from jax.experimental import pallas as pl
from jax.experimental.pallas import tpu as pltpu

_PAR = pltpu.CompilerParams(dimension_semantics=("parallel",))

N = 4096
C = 16
RA = 1024
RB = 1024


def _cast_body(l_ref, x_ref, y_ref, lb_ref):
    l = l_ref[...]
    y_ref[...] = jnp.dot(l, x_ref[...], preferred_element_type=jnp.float32)
    lb_ref[...] = l.astype(jnp.bfloat16)


def _cast_stage(L, x):
    return pl.pallas_call(
        _cast_body,
        grid=(N // RA,),
        in_specs=[
            pl.BlockSpec((RA, N), lambda i: (i, 0)),
            pl.BlockSpec((N, C), lambda i: (0, 0)),
        ],
        out_specs=[
            pl.BlockSpec((RA, C), lambda i: (i, 0)),
            pl.BlockSpec((RA, N), lambda i: (i, 0)),
        ],
        out_shape=[
            jax.ShapeDtypeStruct((N, C), jnp.float32),
            jax.ShapeDtypeStruct((N, N), jnp.bfloat16),
        ],
        compiler_params=_PAR,
    )(L, x)


def _pair_body(ld_ref, lu_ref, x_ref, yd_ref, yu_ref):
    x = x_ref[...].astype(jnp.bfloat16)
    yd_ref[...] = jnp.dot(ld_ref[...], x, preferred_element_type=jnp.float32)
    yu_ref[...] = jnp.dot(lu_ref[...], x, preferred_element_type=jnp.float32)


def _pair_stage(Ldb, Lub, h):
    return pl.pallas_call(
        _pair_body,
        grid=(N // RB,),
        in_specs=[
            pl.BlockSpec((RB, N), lambda i: (i, 0)),
            pl.BlockSpec((RB, N), lambda i: (i, 0)),
            pl.BlockSpec((N, C), lambda i: (0, 0)),
        ],
        out_specs=[
            pl.BlockSpec((RB, C), lambda i: (i, 0)),
            pl.BlockSpec((RB, C), lambda i: (i, 0)),
        ],
        out_shape=[
            jax.ShapeDtypeStruct((N, C), jnp.float32),
            jax.ShapeDtypeStruct((N, C), jnp.float32),
        ],
        compiler_params=_PAR,
    )(Ldb, Lub, h)


def _combine_body(ld_ref, lu_ref, xd_ref, xu_ref, x0_ref, g_ref, b_ref,
                  out_ref):
    i = pl.program_id(0)
    rows = pl.ds(i * RB, RB)
    xd = xd_ref[...].astype(jnp.bfloat16)
    xu = xu_ref[...].astype(jnp.bfloat16)
    xd2 = jnp.dot(ld_ref[...], xd, preferred_element_type=jnp.float32)
    xu2 = jnp.dot(lu_ref[...], xu, preferred_element_type=jnp.float32)
    acc = jnp.dot(x0_ref[rows, :], g_ref[0], preferred_element_type=jnp.float32)
    acc += jnp.dot(xd_ref[rows, :], g_ref[1], preferred_element_type=jnp.float32)
    acc += jnp.dot(xd2, g_ref[2], preferred_element_type=jnp.float32)
    acc += jnp.dot(xu_ref[rows, :], g_ref[3], preferred_element_type=jnp.float32)
    acc += jnp.dot(xu2, g_ref[4], preferred_element_type=jnp.float32)
    out_ref[...] = acc + b_ref[...]


def _combine_stage(Ldb, Lub, xd1, xu1, x0, G, b):
    return pl.pallas_call(
        _combine_body,
        grid=(N // RB,),
        in_specs=[
            pl.BlockSpec((RB, N), lambda i: (i, 0)),
            pl.BlockSpec((RB, N), lambda i: (i, 0)),
            pl.BlockSpec((N, C), lambda i: (0, 0)),
            pl.BlockSpec((N, C), lambda i: (0, 0)),
            pl.BlockSpec((N, C), lambda i: (0, 0)),
            pl.BlockSpec((5, C, C), lambda i: (0, 0, 0)),
            pl.BlockSpec((1, C), lambda i: (0, 0)),
        ],
        out_specs=pl.BlockSpec((RB, C), lambda i: (i, 0)),
        out_shape=jax.ShapeDtypeStruct((N, C), jnp.float32),
        compiler_params=_PAR,
    )(Ldb, Lub, xd1, xu1, x0, G, b)


def kernel(x, laplacian_down, laplacian_up, W1, W2, W_lin, b_lin):
    G1 = jnp.transpose(W1, (2, 0, 1))
    V2 = jnp.einsum("iok,oj->kij", W2, W_lin)
    zero_b = jnp.zeros((1, C), jnp.float32)
    b2 = b_lin.reshape(1, C).astype(jnp.float32)

    xd1, Ldb = _cast_stage(laplacian_down, x)
    xu1, Lub = _cast_stage(laplacian_up, x)
    h = _combine_stage(Ldb, Lub, xd1, xu1, x, G1, zero_b)
    hd1, hu1 = _pair_stage(Ldb, Lub, h)
    out = _combine_stage(Ldb, Lub, hd1, hu1, h, V2, b2)
    return out


# speedup vs baseline: 1.0311x; 1.0241x over previous
import jax
import jax.numpy as jnp
from jax.experimental import pallas as pl
from jax.experimental.pallas import tpu as pltpu

_PAR = pltpu.CompilerParams(dimension_semantics=("parallel",))

N = 4096
C = 16
RA = 512
RB = 1024


def _cast_body(ld_ref, lu_ref, x_ref, yd_ref, yu_ref, ldb_ref, lub_ref):
    ld = ld_ref[...]
    lu = lu_ref[...]
    x = x_ref[...]
    yd_ref[...] = jnp.dot(ld, x, preferred_element_type=jnp.float32)
    yu_ref[...] = jnp.dot(lu, x, preferred_element_type=jnp.float32)
    ldb_ref[...] = ld.astype(jnp.bfloat16)
    lub_ref[...] = lu.astype(jnp.bfloat16)


def _cast_stage(Ld, Lu, x):
    return pl.pallas_call(
        _cast_body,
        grid=(N // RA,),
        in_specs=[
            pl.BlockSpec((RA, N), lambda i: (i, 0)),
            pl.BlockSpec((RA, N), lambda i: (i, 0)),
            pl.BlockSpec((N, C), lambda i: (0, 0)),
        ],
        out_specs=[
            pl.BlockSpec((RA, C), lambda i: (i, 0)),
            pl.BlockSpec((RA, C), lambda i: (i, 0)),
            pl.BlockSpec((RA, N), lambda i: (i, 0)),
            pl.BlockSpec((RA, N), lambda i: (i, 0)),
        ],
        out_shape=[
            jax.ShapeDtypeStruct((N, C), jnp.float32),
            jax.ShapeDtypeStruct((N, C), jnp.float32),
            jax.ShapeDtypeStruct((N, N), jnp.bfloat16),
            jax.ShapeDtypeStruct((N, N), jnp.bfloat16),
        ],
        compiler_params=_PAR,
    )(Ld, Lu, x)


def _pair_body(ld_ref, lu_ref, x_ref, yd_ref, yu_ref):
    x = x_ref[...].astype(jnp.bfloat16)
    yd_ref[...] = jnp.dot(ld_ref[...], x, preferred_element_type=jnp.float32)
    yu_ref[...] = jnp.dot(lu_ref[...], x, preferred_element_type=jnp.float32)


def _pair_stage(Ldb, Lub, h):
    return pl.pallas_call(
        _pair_body,
        grid=(N // RB,),
        in_specs=[
            pl.BlockSpec((RB, N), lambda i: (i, 0)),
            pl.BlockSpec((RB, N), lambda i: (i, 0)),
            pl.BlockSpec((N, C), lambda i: (0, 0)),
        ],
        out_specs=[
            pl.BlockSpec((RB, C), lambda i: (i, 0)),
            pl.BlockSpec((RB, C), lambda i: (i, 0)),
        ],
        out_shape=[
            jax.ShapeDtypeStruct((N, C), jnp.float32),
            jax.ShapeDtypeStruct((N, C), jnp.float32),
        ],
        compiler_params=_PAR,
    )(Ldb, Lub, h)


def _combine_body(ld_ref, lu_ref, xd_ref, xu_ref, x0_ref, g_ref, b_ref,
                  out_ref):
    i = pl.program_id(0)
    rows = pl.ds(i * RB, RB)
    xd = xd_ref[...].astype(jnp.bfloat16)
    xu = xu_ref[...].astype(jnp.bfloat16)
    xd2 = jnp.dot(ld_ref[...], xd, preferred_element_type=jnp.float32)
    xu2 = jnp.dot(lu_ref[...], xu, preferred_element_type=jnp.float32)
    acc = jnp.dot(x0_ref[rows, :], g_ref[0], preferred_element_type=jnp.float32)
    acc += jnp.dot(xd_ref[rows, :], g_ref[1], preferred_element_type=jnp.float32)
    acc += jnp.dot(xd2, g_ref[2], preferred_element_type=jnp.float32)
    acc += jnp.dot(xu_ref[rows, :], g_ref[3], preferred_element_type=jnp.float32)
    acc += jnp.dot(xu2, g_ref[4], preferred_element_type=jnp.float32)
    out_ref[...] = acc + b_ref[...]


def _combine_stage(Ldb, Lub, xd1, xu1, x0, G, b):
    return pl.pallas_call(
        _combine_body,
        grid=(N // RB,),
        in_specs=[
            pl.BlockSpec((RB, N), lambda i: (i, 0)),
            pl.BlockSpec((RB, N), lambda i: (i, 0)),
            pl.BlockSpec((N, C), lambda i: (0, 0)),
            pl.BlockSpec((N, C), lambda i: (0, 0)),
            pl.BlockSpec((N, C), lambda i: (0, 0)),
            pl.BlockSpec((5, C, C), lambda i: (0, 0, 0)),
            pl.BlockSpec((1, C), lambda i: (0, 0)),
        ],
        out_specs=pl.BlockSpec((RB, C), lambda i: (i, 0)),
        out_shape=jax.ShapeDtypeStruct((N, C), jnp.float32),
        compiler_params=_PAR,
    )(Ldb, Lub, xd1, xu1, x0, G, b)


def kernel(x, laplacian_down, laplacian_up, W1, W2, W_lin, b_lin):
    G1 = jnp.transpose(W1, (2, 0, 1))
    V2 = jnp.einsum("iok,oj->kij", W2, W_lin)
    zero_b = jnp.zeros((1, C), jnp.float32)
    b2 = b_lin.reshape(1, C).astype(jnp.float32)

    xd1, xu1, Ldb, Lub = _cast_stage(laplacian_down, laplacian_up, x)
    h = _combine_stage(Ldb, Lub, xd1, xu1, x, G1, zero_b)
    hd1, hu1 = _pair_stage(Ldb, Lub, h)
    out = _combine_stage(Ldb, Lub, hd1, hu1, h, V2, b2)
    return out
